# Initial kernel scaffold; baseline (speedup 1.0000x reference)
import jax
import jax.numpy as jnp
from jax.experimental import pallas as pl


def kernel(x, edge_index, y, edge_attr, W1a, b1a, W1b, b1b, W2a, b2a, W2b, b2b, Wa, ba, Wb, bb):
    raise NotImplementedError("write your pallas kernel here")


# speedup vs baseline: 2.6429x; 2.6429x over previous
import jax
import jax.numpy as jnp
from jax import lax
from jax.experimental import pallas as pl
from jax.experimental.pallas import tpu as pltpu
from jax.experimental.pallas import tpu_sc as plsc

N = 10000
E = 160000
D = 256
EMB = 128
DE = 16

NPAD = 10240
NC, NS = 2, 16
NW = NC * NS

SB = 80
SJ = E // NS // SB
GB = 80
GJ = 2 * E // NW // GB

def _sc_mesh():
    return plsc.VectorSubcoreMesh(
        core_axis_name="c", subcore_axis_name="s",
        num_cores=NC, num_subcores=NS)


def _segsum_body(tab, srcr, dstr, zeros, out, idxs, idxd, rows, shared, sem):
    c = lax.axis_index("c")
    s = lax.axis_index("s")
    pltpu.sync_copy(zeros, shared.at[pl.ds(s * 640, 640)])
    pltpu.sync_copy(srcr.at[c, s], idxs)
    pltpu.sync_copy(dstr.at[s], idxd)
    plsc.subcore_barrier()

    def body(j, carry):
        pltpu.async_copy(tab.at[idxs.at[j]], rows, sem).wait()
        pltpu.sync_copy(rows, shared.at[idxd.at[j]], add=True)
        return carry

    lax.fori_loop(0, SJ, body, 0)
    plsc.subcore_barrier()
    pltpu.sync_copy(shared.at[pl.ds(s * 640, 640)],
                    out.at[pl.ds(c * NPAD + s * 640, 640)])


def _segsum(tab, srcr, dstr, zeros):
    return pl.kernel(
        _segsum_body,
        out_type=jax.ShapeDtypeStruct((2 * NPAD, 128), jnp.float32),
        mesh=_sc_mesh(),
        scratch_types=[
            pltpu.VMEM((SJ, SB), jnp.int32),
            pltpu.VMEM((SJ, SB), jnp.int32),
            pltpu.VMEM((SB, 128), jnp.float32),
            pltpu.VMEM_SHARED((NPAD, 128), jnp.float32),
            pltpu.SemaphoreType.DMA,
        ],
    )(tab, srcr, dstr, zeros)


def _egather_body(tab, tasks, out, idx, rows, sem):
    w = lax.axis_index("c") * NS + lax.axis_index("s")
    pltpu.sync_copy(tasks.at[w], idx)
    base = w * (GJ * GB)

    def body(j, carry):
        pltpu.async_copy(tab.at[idx.at[j]], rows, sem).wait()
        pltpu.sync_copy(rows, out.at[pl.ds(base + j * GB, GB)])
        return carry

    lax.fori_loop(0, GJ, body, 0)


def _egather(tab, tasks):
    return pl.kernel(
        _egather_body,
        out_type=jax.ShapeDtypeStruct((2 * E, 384), jnp.float32),
        mesh=_sc_mesh(),
        scratch_types=[
            pltpu.VMEM((GJ, GB), jnp.int32),
            pltpu.VMEM((GB, 384), jnp.float32),
            pltpu.SemaphoreType.DMA,
        ],
    )(tab, tasks)


MR = 1000
MNB = N // MR

def _mlp1_body(x_ref, al_ref, ar_ref, wx_ref, bx_ref, wy_ref, by_ref, o_ref):
    m = x_ref[...] + jnp.concatenate([al_ref[...], ar_ref[...]], axis=1)
    h = jnp.maximum(
        jnp.dot(m, wx_ref[...], preferred_element_type=jnp.float32)
        + bx_ref[...], 0.0)
    o = jnp.dot(h, wy_ref[...], preferred_element_type=jnp.float32) + by_ref[...]
    o_ref[...] = jnp.maximum(o, 0.0)


def _mlp1(x, al, ar, wx, bx, wy, by):
    return pl.pallas_call(
        _mlp1_body,
        grid=(MNB,),
        in_specs=[
            pl.BlockSpec((MR, D), lambda i: (i, 0)),
            pl.BlockSpec((MR, EMB), lambda i: (i, 0)),
            pl.BlockSpec((MR, EMB), lambda i: (i, 0)),
            pl.BlockSpec((D, D), lambda i: (0, 0)),
            pl.BlockSpec((1, D), lambda i: (0, 0)),
            pl.BlockSpec((D, D), lambda i: (0, 0)),
            pl.BlockSpec((1, D), lambda i: (0, 0)),
        ],
        out_specs=pl.BlockSpec((MR, D), lambda i: (i, 0)),
        out_shape=jax.ShapeDtypeStruct((N, D), jnp.float32),
    )(x, al, ar, wx, bx, wy, by)


def _l2_body(x_ref, al_ref, ar_ref, wx_ref, bx_ref, wy_ref, by_ref, wa_ref,
             z_ref, pt_ref, pb_ref):
    m = x_ref[...] + jnp.concatenate([al_ref[...], ar_ref[...]], axis=1)
    h = jnp.maximum(
        jnp.dot(m, wx_ref[...], preferred_element_type=jnp.float32)
        + bx_ref[...], 0.0)
    z = jnp.dot(h, wy_ref[...], preferred_element_type=jnp.float32) + by_ref[...]
    z_ref[...] = z
    wa = wa_ref[...]
    pt = jnp.dot(z, wa[:EMB, :], preferred_element_type=jnp.float32)
    pb = jnp.dot(z, wa[EMB:, :], preferred_element_type=jnp.float32)
    pt_ref[...] = jnp.concatenate([pt, z], axis=1)
    pb_ref[...] = jnp.concatenate([pb, z], axis=1)


def _l2(x, al, ar, wx, bx, wy, by, wa):
    return pl.pallas_call(
        _l2_body,
        grid=(MNB,),
        in_specs=[
            pl.BlockSpec((MR, D), lambda i: (i, 0)),
            pl.BlockSpec((MR, EMB), lambda i: (i, 0)),
            pl.BlockSpec((MR, EMB), lambda i: (i, 0)),
            pl.BlockSpec((D, D), lambda i: (0, 0)),
            pl.BlockSpec((1, D), lambda i: (0, 0)),
            pl.BlockSpec((D, EMB), lambda i: (0, 0)),
            pl.BlockSpec((1, EMB), lambda i: (0, 0)),
            pl.BlockSpec((D, D), lambda i: (0, 0)),
        ],
        out_specs=[
            pl.BlockSpec((MR, EMB), lambda i: (i, 0)),
            pl.BlockSpec((MR, D + EMB), lambda i: (i, 0)),
            pl.BlockSpec((MR, D + EMB), lambda i: (i, 0)),
        ],
        out_shape=[
            jax.ShapeDtypeStruct((N, EMB), jnp.float32),
            jax.ShapeDtypeStruct((N, D + EMB), jnp.float32),
            jax.ShapeDtypeStruct((N, D + EMB), jnp.float32),
        ],
    )(x, al, ar, wx, bx, wy, by, wa)


EB = 2000
ENB = E // EB

def _edge_body(gs_ref, gd_ref, yf_ref, ea_ref, wb_ref, ba_ref, bb_ref,
               logit_ref, prob_ref, wh_ref, lt_ref, le_ref, la_ref, acc_ref):
    i = pl.program_id(0)
    gs = gs_ref[...]
    gd = gd_ref[...]
    hidden = jnp.maximum(gs[:, :D] + gd[:, :D] + ba_ref[...], 0.0)
    wh = (jnp.dot(hidden, wb_ref[...], preferred_element_type=jnp.float32)
          + bb_ref[...])
    wh_ref[...] = wh

    zs = gs[:, D:]
    zd = gd[:, D:]
    dif = zs - zd
    lane = lax.broadcasted_iota(jnp.int32, (EB, EMB), 1)
    d2 = jnp.sum(jnp.where(lane < EMB - 1, dif * dif, 0.0), axis=1,
                 keepdims=True)
    mass = zd[:, EMB - 1:EMB]
    logits = mass - jnp.log(d2 + 1e-8)
    logit_ref[...] = logits
    prob_ref[...] = jax.nn.sigmoid(logits)

    yf = yf_ref[...]
    bce = (jnp.maximum(logits, 0.0) - logits * yf
           + jnp.log1p(jnp.exp(-jnp.abs(logits))))
    bsum = jnp.sum(bce)
    asum = jnp.sum(((wh - ea_ref[...]) ** 2) * yf)
    csum = jnp.sum(yf)

    @pl.when(i == 0)
    def _():
        acc_ref[0] = bsum
        acc_ref[1] = asum
        acc_ref[2] = csum

    @pl.when(i > 0)
    def _():
        acc_ref[0] = acc_ref[0] + bsum
        acc_ref[1] = acc_ref[1] + asum
        acc_ref[2] = acc_ref[2] + csum

    @pl.when(i == ENB - 1)
    def _():
        le = acc_ref[0] / E
        la = acc_ref[1] / jnp.maximum(acc_ref[2] * DE, 1.0)
        le_ref[...] = jnp.full((1, 1), le, jnp.float32)
        la_ref[...] = jnp.full((1, 1), la, jnp.float32)
        lt_ref[...] = jnp.full((1, 1), le + la, jnp.float32)


def _edge(gs, gd, yf, ea, wb, ba, bb):
    return pl.pallas_call(
        _edge_body,
        grid=(ENB,),
        in_specs=[
            pl.BlockSpec((EB, D + EMB), lambda i: (i, 0)),
            pl.BlockSpec((EB, D + EMB), lambda i: (i, 0)),
            pl.BlockSpec((EB, 1), lambda i: (i, 0)),
            pl.BlockSpec((EB, DE), lambda i: (i, 0)),
            pl.BlockSpec((D, DE), lambda i: (0, 0)),
            pl.BlockSpec((1, D), lambda i: (0, 0)),
            pl.BlockSpec((1, DE), lambda i: (0, 0)),
        ],
        out_specs=[
            pl.BlockSpec((EB, 1), lambda i: (i, 0)),
            pl.BlockSpec((EB, 1), lambda i: (i, 0)),
            pl.BlockSpec((EB, DE), lambda i: (i, 0)),
            pl.BlockSpec((1, 1), lambda i: (0, 0)),
            pl.BlockSpec((1, 1), lambda i: (0, 0)),
            pl.BlockSpec((1, 1), lambda i: (0, 0)),
        ],
        out_shape=[
            jax.ShapeDtypeStruct((E, 1), jnp.float32),
            jax.ShapeDtypeStruct((E, 1), jnp.float32),
            jax.ShapeDtypeStruct((E, DE), jnp.float32),
            jax.ShapeDtypeStruct((1, 1), jnp.float32),
            jax.ShapeDtypeStruct((1, 1), jnp.float32),
            jax.ShapeDtypeStruct((1, 1), jnp.float32),
        ],
        scratch_shapes=[pltpu.SMEM((3,), jnp.float32)],
        compiler_params=pltpu.CompilerParams(
            dimension_semantics=("arbitrary",)),
    )(gs, gd, yf, ea, wb, ba, bb)


def _split_stack(a):
    pad = ((0, NPAD - N), (0, 0))
    return jnp.concatenate(
        [jnp.pad(a[:, :EMB], pad), jnp.pad(a[:, EMB:], pad)], axis=0)


def kernel(x, edge_index, y, edge_attr, W1a, b1a, W1b, b1b, W2a, b2a, W2b,
           b2b, Wa, ba, Wb, bb):
    src = edge_index[0]
    dst = edge_index[1]

    src_c = src.reshape(NS, SJ, SB)
    srcr = jnp.stack([src_c, src_c + NPAD])
    dstr = dst.reshape(NS, SJ, SB)
    tasks = jnp.concatenate([src, dst + NPAD]).reshape(NW, GJ, GB)
    zeros = jnp.zeros((640, EMB), jnp.float32)

    o1 = _segsum(_split_stack(x), srcr, dstr, zeros)
    h1 = _mlp1(x, o1[:N], o1[NPAD:NPAD + N],
               W1a, b1a.reshape(1, -1), W1b, b1b.reshape(1, -1))

    o2 = _segsum(_split_stack(h1), srcr, dstr, zeros)
    z, ptz, pbz = _l2(h1, o2[:N], o2[NPAD:NPAD + N],
                      W2a, b2a.reshape(1, -1), W2b, b2b.reshape(1, -1), Wa)

    pad = ((0, NPAD - N), (0, 0))
    tab3 = jnp.concatenate([jnp.pad(ptz, pad), jnp.pad(pbz, pad)], axis=0)
    g = _egather(tab3, tasks)

    yf = y.astype(jnp.float32).reshape(E, 1)
    logits, prob, wh, lt, le, la = _edge(
        g[:E], g[E:], yf, edge_attr, Wb,
        ba.reshape(1, -1), bb.reshape(1, -1))

    return (lt[0, 0], le[0, 0], la[0, 0], logits[:, 0], prob[:, 0], z, wh)


# speedup vs baseline: 2.7223x; 1.0300x over previous
import jax
import jax.numpy as jnp
from jax import lax
from jax.experimental import pallas as pl
from jax.experimental.pallas import tpu as pltpu
from jax.experimental.pallas import tpu_sc as plsc

N = 10000
E = 160000
D = 256
EMB = 128
DE = 16

NPAD = 10240
NC, NS = 2, 16
NW = NC * NS

SB = 80
SJ = E // NS // SB
GB = 40
GJ = 2 * E // NW // GB
KPS = 1
KPG = 5

def _sc_mesh():
    return plsc.VectorSubcoreMesh(
        core_axis_name="c", subcore_axis_name="s",
        num_cores=NC, num_subcores=NS)


def _segsum_body(tab, srcr, dstr, zeros, out, idxs, idxd, rows, shared, sems):
    c = lax.axis_index("c")
    s = lax.axis_index("s")
    pltpu.sync_copy(zeros, shared.at[pl.ds(s * 640, 640)])
    pltpu.sync_copy(srcr.at[c, s], idxs)
    pltpu.sync_copy(dstr.at[s], idxd)
    plsc.subcore_barrier()

    def body(jj, carry):
        j0 = jj * KPS
        cps = [pltpu.async_copy(tab.at[idxs.at[j0 + k]], rows[k], sems[k])
               for k in range(KPS)]
        for k in range(KPS):
            cps[k].wait()
            pltpu.sync_copy(rows[k], shared.at[idxd.at[j0 + k]], add=True)
        return carry

    lax.fori_loop(0, SJ // KPS, body, 0)
    plsc.subcore_barrier()
    pltpu.sync_copy(shared.at[pl.ds(s * 640, 640)],
                    out.at[pl.ds(c * NPAD + s * 640, 640)])


def _segsum(tab, srcr, dstr, zeros):
    return pl.kernel(
        _segsum_body,
        out_type=jax.ShapeDtypeStruct((2 * NPAD, 128), jnp.float32),
        mesh=_sc_mesh(),
        scratch_types=[
            pltpu.VMEM((SJ, SB), jnp.int32),
            pltpu.VMEM((SJ, SB), jnp.int32),
            [pltpu.VMEM((SB, 128), jnp.float32)] * KPS,
            pltpu.VMEM_SHARED((NPAD, 128), jnp.float32),
            [pltpu.SemaphoreType.DMA] * KPS,
        ],
    )(tab, srcr, dstr, zeros)


def _egather_body(tab, tasks, out, idx, rows, sems):
    w = lax.axis_index("c") * NS + lax.axis_index("s")
    pltpu.sync_copy(tasks.at[w], idx)
    base = w * (GJ * GB)

    def body(jj, carry):
        j0 = jj * KPG
        cps = [pltpu.async_copy(tab.at[idx.at[j0 + k]], rows[k], sems[k])
               for k in range(KPG)]
        for k in range(KPG):
            cps[k].wait()
            pltpu.sync_copy(rows[k],
                            out.at[pl.ds(base + (j0 + k) * GB, GB)])
        return carry

    lax.fori_loop(0, GJ // KPG, body, 0)


def _egather(tab, tasks):
    return pl.kernel(
        _egather_body,
        out_type=jax.ShapeDtypeStruct((2 * E, 384), jnp.float32),
        mesh=_sc_mesh(),
        scratch_types=[
            pltpu.VMEM((GJ, GB), jnp.int32),
            [pltpu.VMEM((GB, 384), jnp.float32)] * KPG,
            [pltpu.SemaphoreType.DMA] * KPG,
        ],
    )(tab, tasks)


MR = 1000
MNB = N // MR

def _mlp1_body(x_ref, al_ref, ar_ref, wx_ref, bx_ref, wy_ref, by_ref, o_ref):
    m = x_ref[...] + jnp.concatenate([al_ref[...], ar_ref[...]], axis=1)
    h = jnp.maximum(
        jnp.dot(m, wx_ref[...], preferred_element_type=jnp.float32)
        + bx_ref[...], 0.0)
    o = jnp.dot(h, wy_ref[...], preferred_element_type=jnp.float32) + by_ref[...]
    o_ref[...] = jnp.maximum(o, 0.0)


def _mlp1(x, al, ar, wx, bx, wy, by):
    return pl.pallas_call(
        _mlp1_body,
        grid=(MNB,),
        in_specs=[
            pl.BlockSpec((MR, D), lambda i: (i, 0)),
            pl.BlockSpec((MR, EMB), lambda i: (i, 0)),
            pl.BlockSpec((MR, EMB), lambda i: (i, 0)),
            pl.BlockSpec((D, D), lambda i: (0, 0)),
            pl.BlockSpec((1, D), lambda i: (0, 0)),
            pl.BlockSpec((D, D), lambda i: (0, 0)),
            pl.BlockSpec((1, D), lambda i: (0, 0)),
        ],
        out_specs=pl.BlockSpec((MR, D), lambda i: (i, 0)),
        out_shape=jax.ShapeDtypeStruct((N, D), jnp.float32),
    )(x, al, ar, wx, bx, wy, by)


def _l2_body(x_ref, al_ref, ar_ref, wx_ref, bx_ref, wy_ref, by_ref, wa_ref,
             z_ref, pt_ref, pb_ref):
    m = x_ref[...] + jnp.concatenate([al_ref[...], ar_ref[...]], axis=1)
    h = jnp.maximum(
        jnp.dot(m, wx_ref[...], preferred_element_type=jnp.float32)
        + bx_ref[...], 0.0)
    z = jnp.dot(h, wy_ref[...], preferred_element_type=jnp.float32) + by_ref[...]
    z_ref[...] = z
    wa = wa_ref[...]
    pt = jnp.dot(z, wa[:EMB, :], preferred_element_type=jnp.float32)
    pb = jnp.dot(z, wa[EMB:, :], preferred_element_type=jnp.float32)
    pt_ref[...] = jnp.concatenate([pt, z], axis=1)
    pb_ref[...] = jnp.concatenate([pb, z], axis=1)


def _l2(x, al, ar, wx, bx, wy, by, wa):
    return pl.pallas_call(
        _l2_body,
        grid=(MNB,),
        in_specs=[
            pl.BlockSpec((MR, D), lambda i: (i, 0)),
            pl.BlockSpec((MR, EMB), lambda i: (i, 0)),
            pl.BlockSpec((MR, EMB), lambda i: (i, 0)),
            pl.BlockSpec((D, D), lambda i: (0, 0)),
            pl.BlockSpec((1, D), lambda i: (0, 0)),
            pl.BlockSpec((D, EMB), lambda i: (0, 0)),
            pl.BlockSpec((1, EMB), lambda i: (0, 0)),
            pl.BlockSpec((D, D), lambda i: (0, 0)),
        ],
        out_specs=[
            pl.BlockSpec((MR, EMB), lambda i: (i, 0)),
            pl.BlockSpec((MR, D + EMB), lambda i: (i, 0)),
            pl.BlockSpec((MR, D + EMB), lambda i: (i, 0)),
        ],
        out_shape=[
            jax.ShapeDtypeStruct((N, EMB), jnp.float32),
            jax.ShapeDtypeStruct((N, D + EMB), jnp.float32),
            jax.ShapeDtypeStruct((N, D + EMB), jnp.float32),
        ],
    )(x, al, ar, wx, bx, wy, by, wa)


EB = 2000
ENB = E // EB

def _edge_body(gs_ref, gd_ref, yf_ref, ea_ref, wb_ref, ba_ref, bb_ref,
               logit_ref, prob_ref, wh_ref, lt_ref, le_ref, la_ref, acc_ref):
    i = pl.program_id(0)
    gs = gs_ref[...]
    gd = gd_ref[...]
    hidden = jnp.maximum(gs[:, :D] + gd[:, :D] + ba_ref[...], 0.0)
    wh = (jnp.dot(hidden, wb_ref[...], preferred_element_type=jnp.float32)
          + bb_ref[...])
    wh_ref[...] = wh

    zs = gs[:, D:]
    zd = gd[:, D:]
    dif = zs - zd
    lane = lax.broadcasted_iota(jnp.int32, (EB, EMB), 1)
    d2 = jnp.sum(jnp.where(lane < EMB - 1, dif * dif, 0.0), axis=1,
                 keepdims=True)
    mass = zd[:, EMB - 1:EMB]
    logits = mass - jnp.log(d2 + 1e-8)
    logit_ref[...] = logits
    prob_ref[...] = jax.nn.sigmoid(logits)

    yf = yf_ref[...]
    bce = (jnp.maximum(logits, 0.0) - logits * yf
           + jnp.log1p(jnp.exp(-jnp.abs(logits))))
    bsum = jnp.sum(bce)
    asum = jnp.sum(((wh - ea_ref[...]) ** 2) * yf)
    csum = jnp.sum(yf)

    @pl.when(i == 0)
    def _():
        acc_ref[0] = bsum
        acc_ref[1] = asum
        acc_ref[2] = csum

    @pl.when(i > 0)
    def _():
        acc_ref[0] = acc_ref[0] + bsum
        acc_ref[1] = acc_ref[1] + asum
        acc_ref[2] = acc_ref[2] + csum

    @pl.when(i == ENB - 1)
    def _():
        le = acc_ref[0] / E
        la = acc_ref[1] / jnp.maximum(acc_ref[2] * DE, 1.0)
        le_ref[...] = jnp.full((1, 1), le, jnp.float32)
        la_ref[...] = jnp.full((1, 1), la, jnp.float32)
        lt_ref[...] = jnp.full((1, 1), le + la, jnp.float32)


def _edge(gs, gd, yf, ea, wb, ba, bb):
    return pl.pallas_call(
        _edge_body,
        grid=(ENB,),
        in_specs=[
            pl.BlockSpec((EB, D + EMB), lambda i: (i, 0)),
            pl.BlockSpec((EB, D + EMB), lambda i: (i, 0)),
            pl.BlockSpec((EB, 1), lambda i: (i, 0)),
            pl.BlockSpec((EB, DE), lambda i: (i, 0)),
            pl.BlockSpec((D, DE), lambda i: (0, 0)),
            pl.BlockSpec((1, D), lambda i: (0, 0)),
            pl.BlockSpec((1, DE), lambda i: (0, 0)),
        ],
        out_specs=[
            pl.BlockSpec((EB, 1), lambda i: (i, 0)),
            pl.BlockSpec((EB, 1), lambda i: (i, 0)),
            pl.BlockSpec((EB, DE), lambda i: (i, 0)),
            pl.BlockSpec((1, 1), lambda i: (0, 0)),
            pl.BlockSpec((1, 1), lambda i: (0, 0)),
            pl.BlockSpec((1, 1), lambda i: (0, 0)),
        ],
        out_shape=[
            jax.ShapeDtypeStruct((E, 1), jnp.float32),
            jax.ShapeDtypeStruct((E, 1), jnp.float32),
            jax.ShapeDtypeStruct((E, DE), jnp.float32),
            jax.ShapeDtypeStruct((1, 1), jnp.float32),
            jax.ShapeDtypeStruct((1, 1), jnp.float32),
            jax.ShapeDtypeStruct((1, 1), jnp.float32),
        ],
        scratch_shapes=[pltpu.SMEM((3,), jnp.float32)],
        compiler_params=pltpu.CompilerParams(
            dimension_semantics=("arbitrary",)),
    )(gs, gd, yf, ea, wb, ba, bb)


def _split_stack(a):
    pad = ((0, NPAD - N), (0, 0))
    return jnp.concatenate(
        [jnp.pad(a[:, :EMB], pad), jnp.pad(a[:, EMB:], pad)], axis=0)


def kernel(x, edge_index, y, edge_attr, W1a, b1a, W1b, b1b, W2a, b2a, W2b,
           b2b, Wa, ba, Wb, bb):
    src = edge_index[0]
    dst = edge_index[1]

    src_c = src.reshape(NS, SJ, SB)
    srcr = jnp.stack([src_c, src_c + NPAD])
    dstr = dst.reshape(NS, SJ, SB)
    tasks = jnp.concatenate([src, dst + NPAD]).reshape(NW, GJ, GB)
    zeros = jnp.zeros((640, EMB), jnp.float32)

    o1 = _segsum(_split_stack(x), srcr, dstr, zeros)
    h1 = _mlp1(x, o1[:N], o1[NPAD:NPAD + N],
               W1a, b1a.reshape(1, -1), W1b, b1b.reshape(1, -1))

    o2 = _segsum(_split_stack(h1), srcr, dstr, zeros)
    z, ptz, pbz = _l2(h1, o2[:N], o2[NPAD:NPAD + N],
                      W2a, b2a.reshape(1, -1), W2b, b2b.reshape(1, -1), Wa)

    pad = ((0, NPAD - N), (0, 0))
    tab3 = jnp.concatenate([jnp.pad(ptz, pad), jnp.pad(pbz, pad)], axis=0)
    g = _egather(tab3, tasks)

    yf = y.astype(jnp.float32).reshape(E, 1)
    logits, prob, wh, lt, le, la = _edge(
        g[:E], g[E:], yf, edge_attr, Wb,
        ba.reshape(1, -1), bb.reshape(1, -1))

    return (lt[0, 0], le[0, 0], la[0, 0], logits[:, 0], prob[:, 0], z, wh)


# speedup vs baseline: 3.2151x; 1.1810x over previous
import jax
import jax.numpy as jnp
from jax import lax
from jax.experimental import pallas as pl
from jax.experimental.pallas import tpu as pltpu
from jax.experimental.pallas import tpu_sc as plsc

N = 10000
E = 160000
D = 256
EMB = 128
DE = 16

NPAD = 10240
NC, NS = 2, 16
NW = NC * NS

SB = 80
SJ = E // NS // SB
GB = 80
GJ = 2 * E // NW // GB
KPS = 1
KPG = 5

def _sc_mesh():
    return plsc.VectorSubcoreMesh(
        core_axis_name="c", subcore_axis_name="s",
        num_cores=NC, num_subcores=NS)


def _segsum_body(tab, srcr, dstr, zeros, out, idxs, idxd, rows, shared, sems):
    c = lax.axis_index("c")
    s = lax.axis_index("s")
    pltpu.sync_copy(zeros, shared.at[pl.ds(s * 640, 640)])
    pltpu.sync_copy(srcr.at[c, s], idxs)
    pltpu.sync_copy(dstr.at[s], idxd)
    plsc.subcore_barrier()

    def body(jj, carry):
        j0 = jj * KPS
        cps = [pltpu.async_copy(tab.at[idxs.at[j0 + k]], rows[k], sems[k])
               for k in range(KPS)]
        for k in range(KPS):
            cps[k].wait()
            pltpu.sync_copy(rows[k], shared.at[idxd.at[j0 + k]], add=True)
        return carry

    lax.fori_loop(0, SJ // KPS, body, 0)
    plsc.subcore_barrier()
    pltpu.sync_copy(shared.at[pl.ds(s * 640, 640)],
                    out.at[pl.ds(c * NPAD + s * 640, 640)])


def _segsum(tab, srcr, dstr, zeros):
    return pl.kernel(
        _segsum_body,
        out_type=jax.ShapeDtypeStruct((2 * NPAD, 128), jnp.float32),
        mesh=_sc_mesh(),
        scratch_types=[
            pltpu.VMEM((SJ, SB), jnp.int32),
            pltpu.VMEM((SJ, SB), jnp.int32),
            [pltpu.VMEM((SB, 128), jnp.float32)] * KPS,
            pltpu.VMEM_SHARED((NPAD, 128), jnp.float32),
            [pltpu.SemaphoreType.DMA] * KPS,
        ],
    )(tab, srcr, dstr, zeros)


def _egather_body(tab, tasks, out, idx, rows, sems):
    w = lax.axis_index("c") * NS + lax.axis_index("s")
    pltpu.sync_copy(tasks.at[w], idx)
    base = w * (GJ * GB)

    def body(jj, carry):
        j0 = jj * KPG
        cps = [pltpu.async_copy(tab.at[idx.at[j0 + k]], rows[k], sems[k])
               for k in range(KPG)]
        for k in range(KPG):
            cps[k].wait()
            pltpu.sync_copy(rows[k],
                            out.at[pl.ds(base + (j0 + k) * GB, GB)])
        return carry

    lax.fori_loop(0, GJ // KPG, body, 0)


def _egather(tab, tasks):
    return pl.kernel(
        _egather_body,
        out_type=jax.ShapeDtypeStruct((2 * E, PW), jnp.uint32),
        mesh=_sc_mesh(),
        scratch_types=[
            pltpu.VMEM((GJ, GB), jnp.int32),
            [pltpu.VMEM((GB, PW), jnp.uint32)] * KPG,
            [pltpu.SemaphoreType.DMA] * KPG,
        ],
    )(tab, tasks)


MR = 1000
MNB = N // MR

def _mlp1_body(x_ref, al_ref, ar_ref, wx_ref, bx_ref, wy_ref, by_ref, o_ref):
    m = x_ref[...] + jnp.concatenate([al_ref[...], ar_ref[...]], axis=1)
    h = jnp.maximum(
        jnp.dot(m, wx_ref[...], preferred_element_type=jnp.float32)
        + bx_ref[...], 0.0)
    o = jnp.dot(h, wy_ref[...], preferred_element_type=jnp.float32) + by_ref[...]
    o_ref[...] = jnp.maximum(o, 0.0)


def _mlp1(x, al, ar, wx, bx, wy, by):
    return pl.pallas_call(
        _mlp1_body,
        grid=(MNB,),
        in_specs=[
            pl.BlockSpec((MR, D), lambda i: (i, 0)),
            pl.BlockSpec((MR, EMB), lambda i: (i, 0)),
            pl.BlockSpec((MR, EMB), lambda i: (i, 0)),
            pl.BlockSpec((D, D), lambda i: (0, 0)),
            pl.BlockSpec((1, D), lambda i: (0, 0)),
            pl.BlockSpec((D, D), lambda i: (0, 0)),
            pl.BlockSpec((1, D), lambda i: (0, 0)),
        ],
        out_specs=pl.BlockSpec((MR, D), lambda i: (i, 0)),
        out_shape=jax.ShapeDtypeStruct((N, D), jnp.float32),
    )(x, al, ar, wx, bx, wy, by)


PW = D // 2 + EMB


def _pack(p, z):
    bits = lax.bitcast_convert_type(p, jnp.uint32) + jnp.uint32(0x8000)
    lo = bits[:, :D // 2] >> 16
    hi = bits[:, D // 2:] & jnp.uint32(0xFFFF0000)
    return jnp.concatenate(
        [lo | hi, lax.bitcast_convert_type(z, jnp.uint32)], axis=1)


def _l2_body(x_ref, al_ref, ar_ref, wx_ref, bx_ref, wy_ref, by_ref, wa_ref,
             z_ref, pt_ref, pb_ref):
    m = x_ref[...] + jnp.concatenate([al_ref[...], ar_ref[...]], axis=1)
    h = jnp.maximum(
        jnp.dot(m, wx_ref[...], preferred_element_type=jnp.float32)
        + bx_ref[...], 0.0)
    z = jnp.dot(h, wy_ref[...], preferred_element_type=jnp.float32) + by_ref[...]
    z_ref[...] = z
    wa = wa_ref[...]
    pt = jnp.dot(z, wa[:EMB, :], preferred_element_type=jnp.float32)
    pb = jnp.dot(z, wa[EMB:, :], preferred_element_type=jnp.float32)
    pt_ref[...] = _pack(pt, z)
    pb_ref[...] = _pack(pb, z)


def _l2(x, al, ar, wx, bx, wy, by, wa):
    return pl.pallas_call(
        _l2_body,
        grid=(MNB,),
        in_specs=[
            pl.BlockSpec((MR, D), lambda i: (i, 0)),
            pl.BlockSpec((MR, EMB), lambda i: (i, 0)),
            pl.BlockSpec((MR, EMB), lambda i: (i, 0)),
            pl.BlockSpec((D, D), lambda i: (0, 0)),
            pl.BlockSpec((1, D), lambda i: (0, 0)),
            pl.BlockSpec((D, EMB), lambda i: (0, 0)),
            pl.BlockSpec((1, EMB), lambda i: (0, 0)),
            pl.BlockSpec((D, D), lambda i: (0, 0)),
        ],
        out_specs=[
            pl.BlockSpec((MR, EMB), lambda i: (i, 0)),
            pl.BlockSpec((MR, PW), lambda i: (i, 0)),
            pl.BlockSpec((MR, PW), lambda i: (i, 0)),
        ],
        out_shape=[
            jax.ShapeDtypeStruct((N, EMB), jnp.float32),
            jax.ShapeDtypeStruct((N, PW), jnp.uint32),
            jax.ShapeDtypeStruct((N, PW), jnp.uint32),
        ],
    )(x, al, ar, wx, bx, wy, by, wa)


EB = 2000
ENB = E // EB


def _unpack(w):
    wp = w[:, :D // 2]
    lo = lax.bitcast_convert_type(wp << 16, jnp.float32)
    hi = lax.bitcast_convert_type(wp & jnp.uint32(0xFFFF0000), jnp.float32)
    z = lax.bitcast_convert_type(w[:, D // 2:], jnp.float32)
    return jnp.concatenate([lo, hi], axis=1), z

def _edge_body(gs_ref, gd_ref, yf_ref, ea_ref, wb_ref, ba_ref, bb_ref,
               logit_ref, prob_ref, wh_ref, lt_ref, le_ref, la_ref, acc_ref):
    i = pl.program_id(0)
    ps, zs = _unpack(gs_ref[...])
    pd, zd = _unpack(gd_ref[...])
    hidden = jnp.maximum(ps + pd + ba_ref[...], 0.0)
    wh = (jnp.dot(hidden, wb_ref[...], preferred_element_type=jnp.float32)
          + bb_ref[...])
    wh_ref[...] = wh

    dif = zs - zd
    lane = lax.broadcasted_iota(jnp.int32, (EB, EMB), 1)
    d2 = jnp.sum(jnp.where(lane < EMB - 1, dif * dif, 0.0), axis=1,
                 keepdims=True)
    mass = zd[:, EMB - 1:EMB]
    logits = mass - jnp.log(d2 + 1e-8)
    logit_ref[...] = logits
    prob_ref[...] = jax.nn.sigmoid(logits)

    yf = yf_ref[...]
    bce = (jnp.maximum(logits, 0.0) - logits * yf
           + jnp.log1p(jnp.exp(-jnp.abs(logits))))
    bsum = jnp.sum(bce)
    asum = jnp.sum(((wh - ea_ref[...]) ** 2) * yf)
    csum = jnp.sum(yf)

    @pl.when(i == 0)
    def _():
        acc_ref[0] = bsum
        acc_ref[1] = asum
        acc_ref[2] = csum

    @pl.when(i > 0)
    def _():
        acc_ref[0] = acc_ref[0] + bsum
        acc_ref[1] = acc_ref[1] + asum
        acc_ref[2] = acc_ref[2] + csum

    @pl.when(i == ENB - 1)
    def _():
        le = acc_ref[0] / E
        la = acc_ref[1] / jnp.maximum(acc_ref[2] * DE, 1.0)
        le_ref[...] = jnp.full((1, 1), le, jnp.float32)
        la_ref[...] = jnp.full((1, 1), la, jnp.float32)
        lt_ref[...] = jnp.full((1, 1), le + la, jnp.float32)


def _edge(gs, gd, yf, ea, wb, ba, bb):
    return pl.pallas_call(
        _edge_body,
        grid=(ENB,),
        in_specs=[
            pl.BlockSpec((EB, PW), lambda i: (i, 0)),
            pl.BlockSpec((EB, PW), lambda i: (i, 0)),
            pl.BlockSpec((EB, 1), lambda i: (i, 0)),
            pl.BlockSpec((EB, DE), lambda i: (i, 0)),
            pl.BlockSpec((D, DE), lambda i: (0, 0)),
            pl.BlockSpec((1, D), lambda i: (0, 0)),
            pl.BlockSpec((1, DE), lambda i: (0, 0)),
        ],
        out_specs=[
            pl.BlockSpec((EB, 1), lambda i: (i, 0)),
            pl.BlockSpec((EB, 1), lambda i: (i, 0)),
            pl.BlockSpec((EB, DE), lambda i: (i, 0)),
            pl.BlockSpec((1, 1), lambda i: (0, 0)),
            pl.BlockSpec((1, 1), lambda i: (0, 0)),
            pl.BlockSpec((1, 1), lambda i: (0, 0)),
        ],
        out_shape=[
            jax.ShapeDtypeStruct((E, 1), jnp.float32),
            jax.ShapeDtypeStruct((E, 1), jnp.float32),
            jax.ShapeDtypeStruct((E, DE), jnp.float32),
            jax.ShapeDtypeStruct((1, 1), jnp.float32),
            jax.ShapeDtypeStruct((1, 1), jnp.float32),
            jax.ShapeDtypeStruct((1, 1), jnp.float32),
        ],
        scratch_shapes=[pltpu.SMEM((3,), jnp.float32)],
        compiler_params=pltpu.CompilerParams(
            dimension_semantics=("arbitrary",)),
    )(gs, gd, yf, ea, wb, ba, bb)


def _split_stack(a):
    pad = ((0, NPAD - N), (0, 0))
    return jnp.concatenate(
        [jnp.pad(a[:, :EMB], pad), jnp.pad(a[:, EMB:], pad)], axis=0)


def kernel(x, edge_index, y, edge_attr, W1a, b1a, W1b, b1b, W2a, b2a, W2b,
           b2b, Wa, ba, Wb, bb):
    src = edge_index[0]
    dst = edge_index[1]

    src_c = src.reshape(NS, SJ, SB)
    srcr = jnp.stack([src_c, src_c + NPAD])
    dstr = dst.reshape(NS, SJ, SB)
    tasks = jnp.concatenate([src, dst + NPAD]).reshape(NW, GJ, GB)
    zeros = jnp.zeros((640, EMB), jnp.float32)

    o1 = _segsum(_split_stack(x), srcr, dstr, zeros)
    h1 = _mlp1(x, o1[:N], o1[NPAD:NPAD + N],
               W1a, b1a.reshape(1, -1), W1b, b1b.reshape(1, -1))

    o2 = _segsum(_split_stack(h1), srcr, dstr, zeros)
    z, ptz, pbz = _l2(h1, o2[:N], o2[NPAD:NPAD + N],
                      W2a, b2a.reshape(1, -1), W2b, b2b.reshape(1, -1), Wa)

    pad = ((0, NPAD - N), (0, 0))
    tab3 = jnp.concatenate([jnp.pad(ptz, pad), jnp.pad(pbz, pad)], axis=0)
    g = _egather(tab3, tasks)

    yf = y.astype(jnp.float32).reshape(E, 1)
    logits, prob, wh, lt, le, la = _edge(
        g[:E], g[E:], yf, edge_attr, Wb,
        ba.reshape(1, -1), bb.reshape(1, -1))

    return (lt[0, 0], le[0, 0], la[0, 0], logits[:, 0], prob[:, 0], z, wh)


# speedup vs baseline: 3.7812x; 1.1761x over previous
import jax
import jax.numpy as jnp
from jax import lax
from jax.experimental import pallas as pl
from jax.experimental.pallas import tpu as pltpu
from jax.experimental.pallas import tpu_sc as plsc

N = 10000
E = 160000
D = 256
EMB = 128
DE = 16

NPAD = 10240
NC, NS = 2, 16
NW = NC * NS

SB = 80
SJ = E // NS // SB
GB = 80
GJ = 2 * E // NW // GB
KPS = 1
KPG = 5

def _sc_mesh():
    return plsc.VectorSubcoreMesh(
        core_axis_name="c", subcore_axis_name="s",
        num_cores=NC, num_subcores=NS)


def _segsum_body(tab, srcr, dstr, zeros, out, idxs, idxd, rows, shared, sems):
    c = lax.axis_index("c")
    s = lax.axis_index("s")
    pltpu.sync_copy(zeros, shared.at[pl.ds(s * 640, 640)])
    pltpu.sync_copy(srcr.at[c, s], idxs)
    pltpu.sync_copy(dstr.at[s], idxd)
    plsc.subcore_barrier()

    def body(jj, carry):
        j0 = jj * KPS
        cps = [pltpu.async_copy(tab.at[idxs.at[j0 + k]], rows[k], sems[k])
               for k in range(KPS)]
        for k in range(KPS):
            cps[k].wait()
            pltpu.sync_copy(rows[k], shared.at[idxd.at[j0 + k]], add=True)
        return carry

    lax.fori_loop(0, SJ // KPS, body, 0)
    plsc.subcore_barrier()
    pltpu.sync_copy(shared.at[pl.ds(s * 640, 640)],
                    out.at[pl.ds(c * NPAD + s * 640, 640)])


def _segsum(tab, srcr, dstr, zeros):
    return pl.kernel(
        _segsum_body,
        out_type=jax.ShapeDtypeStruct((2 * NPAD, 128), jnp.float32),
        mesh=_sc_mesh(),
        scratch_types=[
            pltpu.VMEM((SJ, SB), jnp.int32),
            pltpu.VMEM((SJ, SB), jnp.int32),
            [pltpu.VMEM((SB, 128), jnp.float32)] * KPS,
            pltpu.VMEM_SHARED((NPAD, 128), jnp.float32),
            [pltpu.SemaphoreType.DMA] * KPS,
        ],
    )(tab, srcr, dstr, zeros)


def _egather_body(tab, tasks, out, idx, rows, sems):
    w = lax.axis_index("c") * NS + lax.axis_index("s")
    pltpu.sync_copy(tasks.at[w], idx)
    base = w * (GJ * GB)

    def body(jj, carry):
        j0 = jj * KPG
        cps = [pltpu.async_copy(tab.at[idx.at[j0 + k]], rows[k], sems[k])
               for k in range(KPG)]
        for k in range(KPG):
            cps[k].wait()
            pltpu.sync_copy(rows[k],
                            out.at[pl.ds(base + (j0 + k) * GB, GB)])
        return carry

    lax.fori_loop(0, GJ // KPG, body, 0)


def _egather(tab, tasks):
    return pl.kernel(
        _egather_body,
        out_type=jax.ShapeDtypeStruct((2 * E, PW), jnp.uint32),
        mesh=_sc_mesh(),
        scratch_types=[
            pltpu.VMEM((GJ, GB), jnp.int32),
            [pltpu.VMEM((GB, PW), jnp.uint32)] * KPG,
            [pltpu.SemaphoreType.DMA] * KPG,
        ],
    )(tab, tasks)


MR = 1000
MNB = N // MR

def _mlp1_body(x_ref, al_ref, ar_ref, wx_ref, bx_ref, wy_ref, by_ref, o_ref):
    m = x_ref[...] + jnp.concatenate([al_ref[...], ar_ref[...]], axis=1)
    h = jnp.maximum(
        jnp.dot(m, wx_ref[...], preferred_element_type=jnp.float32)
        + bx_ref[...], 0.0)
    o = jnp.dot(h, wy_ref[...], preferred_element_type=jnp.float32) + by_ref[...]
    o_ref[...] = jnp.maximum(o, 0.0)


def _mlp1(x, al, ar, wx, bx, wy, by):
    return pl.pallas_call(
        _mlp1_body,
        grid=(MNB,),
        in_specs=[
            pl.BlockSpec((MR, D), lambda i: (i, 0)),
            pl.BlockSpec((MR, EMB), lambda i: (i, 0)),
            pl.BlockSpec((MR, EMB), lambda i: (i, 0)),
            pl.BlockSpec((D, D), lambda i: (0, 0)),
            pl.BlockSpec((1, D), lambda i: (0, 0)),
            pl.BlockSpec((D, D), lambda i: (0, 0)),
            pl.BlockSpec((1, D), lambda i: (0, 0)),
        ],
        out_specs=pl.BlockSpec((MR, D), lambda i: (i, 0)),
        out_shape=jax.ShapeDtypeStruct((N, D), jnp.float32),
    )(x, al, ar, wx, bx, wy, by)


PW = D // 2 + EMB


def _pack(p, z):
    bits = lax.bitcast_convert_type(p, jnp.uint32) + jnp.uint32(0x8000)
    lo = bits[:, :D // 2] >> 16
    hi = bits[:, D // 2:] & jnp.uint32(0xFFFF0000)
    return jnp.concatenate(
        [lo | hi, lax.bitcast_convert_type(z, jnp.uint32)], axis=1)


def _l2_body(x_ref, al_ref, ar_ref, wx_ref, bx_ref, wy_ref, by_ref, wa_ref,
             z_ref, pt_ref, pb_ref):
    m = x_ref[...] + jnp.concatenate([al_ref[...], ar_ref[...]], axis=1)
    h = jnp.maximum(
        jnp.dot(m, wx_ref[...], preferred_element_type=jnp.float32)
        + bx_ref[...], 0.0)
    z = jnp.dot(h, wy_ref[...], preferred_element_type=jnp.float32) + by_ref[...]
    z_ref[...] = z
    wa = wa_ref[...]
    pt = jnp.dot(z, wa[:EMB, :], preferred_element_type=jnp.float32)
    pb = jnp.dot(z, wa[EMB:, :], preferred_element_type=jnp.float32)
    pt_ref[...] = _pack(pt, z)
    pb_ref[...] = _pack(pb, z)


def _l2(x, al, ar, wx, bx, wy, by, wa):
    return pl.pallas_call(
        _l2_body,
        grid=(MNB,),
        in_specs=[
            pl.BlockSpec((MR, D), lambda i: (i, 0)),
            pl.BlockSpec((MR, EMB), lambda i: (i, 0)),
            pl.BlockSpec((MR, EMB), lambda i: (i, 0)),
            pl.BlockSpec((D, D), lambda i: (0, 0)),
            pl.BlockSpec((1, D), lambda i: (0, 0)),
            pl.BlockSpec((D, EMB), lambda i: (0, 0)),
            pl.BlockSpec((1, EMB), lambda i: (0, 0)),
            pl.BlockSpec((D, D), lambda i: (0, 0)),
        ],
        out_specs=[
            pl.BlockSpec((MR, EMB), lambda i: (i, 0)),
            pl.BlockSpec((MR, PW), lambda i: (i, 0)),
            pl.BlockSpec((MR, PW), lambda i: (i, 0)),
        ],
        out_shape=[
            jax.ShapeDtypeStruct((N, EMB), jnp.float32),
            jax.ShapeDtypeStruct((N, PW), jnp.uint32),
            jax.ShapeDtypeStruct((N, PW), jnp.uint32),
        ],
    )(x, al, ar, wx, bx, wy, by, wa)


EB = 2000
ENB = E // EB


def _unpack(w):
    wp = w[:, :D // 2]
    lo = lax.bitcast_convert_type(wp << 16, jnp.float32)
    hi = lax.bitcast_convert_type(wp & jnp.uint32(0xFFFF0000), jnp.float32)
    z = lax.bitcast_convert_type(w[:, D // 2:], jnp.float32)
    return jnp.concatenate([lo, hi], axis=1), z

def _edge_body(gs_ref, gd_ref, yf_ref, ea_ref, wb_ref, ba_ref, bb_ref,
               logit_ref, prob_ref, wh_ref, lt_ref, le_ref, la_ref, acc_ref):
    i = pl.program_id(0)
    ps, zs = _unpack(gs_ref[...])
    pd, zd = _unpack(gd_ref[...])
    hidden = jnp.maximum(ps + pd + ba_ref[...], 0.0)
    wh = (jnp.dot(hidden, wb_ref[...], preferred_element_type=jnp.float32)
          + bb_ref[...])
    wh_ref[...] = wh

    dif = zs - zd
    lane = lax.broadcasted_iota(jnp.int32, (EB, EMB), 1)
    d2 = jnp.sum(jnp.where(lane < EMB - 1, dif * dif, 0.0), axis=1,
                 keepdims=True)
    mass = zd[:, EMB - 1:EMB]
    logits = mass - jnp.log(d2 + 1e-8)
    logit_ref[...] = logits
    prob_ref[...] = jax.nn.sigmoid(logits)

    yf = yf_ref[...]
    bce = (jnp.maximum(logits, 0.0) - logits * yf
           + jnp.log1p(jnp.exp(-jnp.abs(logits))))
    bsum = jnp.sum(bce)
    asum = jnp.sum(((wh - ea_ref[...]) ** 2) * yf)
    csum = jnp.sum(yf)

    @pl.when(i == 0)
    def _():
        acc_ref[0] = bsum
        acc_ref[1] = asum
        acc_ref[2] = csum

    @pl.when(i > 0)
    def _():
        acc_ref[0] = acc_ref[0] + bsum
        acc_ref[1] = acc_ref[1] + asum
        acc_ref[2] = acc_ref[2] + csum

    @pl.when(i == ENB - 1)
    def _():
        le = acc_ref[0] / E
        la = acc_ref[1] / jnp.maximum(acc_ref[2] * DE, 1.0)
        le_ref[...] = jnp.full((1, 1), le, jnp.float32)
        la_ref[...] = jnp.full((1, 1), la, jnp.float32)
        lt_ref[...] = jnp.full((1, 1), le + la, jnp.float32)


def _edge(g, yf, ea, wb, ba, bb):
    return pl.pallas_call(
        _edge_body,
        grid=(ENB,),
        in_specs=[
            pl.BlockSpec((EB, PW), lambda i: (i, 0)),
            pl.BlockSpec((EB, PW), lambda i: (i + ENB, 0)),
            pl.BlockSpec((EB, 1), lambda i: (i, 0)),
            pl.BlockSpec((EB, DE), lambda i: (i, 0)),
            pl.BlockSpec((D, DE), lambda i: (0, 0)),
            pl.BlockSpec((1, D), lambda i: (0, 0)),
            pl.BlockSpec((1, DE), lambda i: (0, 0)),
        ],
        out_specs=[
            pl.BlockSpec((EB, 1), lambda i: (i, 0)),
            pl.BlockSpec((EB, 1), lambda i: (i, 0)),
            pl.BlockSpec((EB, DE), lambda i: (i, 0)),
            pl.BlockSpec((1, 1), lambda i: (0, 0)),
            pl.BlockSpec((1, 1), lambda i: (0, 0)),
            pl.BlockSpec((1, 1), lambda i: (0, 0)),
        ],
        out_shape=[
            jax.ShapeDtypeStruct((E, 1), jnp.float32),
            jax.ShapeDtypeStruct((E, 1), jnp.float32),
            jax.ShapeDtypeStruct((E, DE), jnp.float32),
            jax.ShapeDtypeStruct((1, 1), jnp.float32),
            jax.ShapeDtypeStruct((1, 1), jnp.float32),
            jax.ShapeDtypeStruct((1, 1), jnp.float32),
        ],
        scratch_shapes=[pltpu.SMEM((3,), jnp.float32)],
        compiler_params=pltpu.CompilerParams(
            dimension_semantics=("arbitrary",)),
    )(g, g, yf, ea, wb, ba, bb)


def _split_stack(a):
    pad = ((0, NPAD - N), (0, 0))
    return jnp.concatenate(
        [jnp.pad(a[:, :EMB], pad), jnp.pad(a[:, EMB:], pad)], axis=0)


def kernel(x, edge_index, y, edge_attr, W1a, b1a, W1b, b1b, W2a, b2a, W2b,
           b2b, Wa, ba, Wb, bb):
    src = edge_index[0]
    dst = edge_index[1]

    src_c = src.reshape(NS, SJ, SB)
    srcr = jnp.stack([src_c, src_c + NPAD])
    dstr = dst.reshape(NS, SJ, SB)
    tasks = jnp.concatenate([src, dst + NPAD]).reshape(NW, GJ, GB)
    zeros = jnp.zeros((640, EMB), jnp.float32)

    o1 = _segsum(_split_stack(x), srcr, dstr, zeros)
    h1 = _mlp1(x, o1[:N], o1[NPAD:NPAD + N],
               W1a, b1a.reshape(1, -1), W1b, b1b.reshape(1, -1))

    o2 = _segsum(_split_stack(h1), srcr, dstr, zeros)
    z, ptz, pbz = _l2(h1, o2[:N], o2[NPAD:NPAD + N],
                      W2a, b2a.reshape(1, -1), W2b, b2b.reshape(1, -1), Wa)

    pad = ((0, NPAD - N), (0, 0))
    tab3 = jnp.concatenate([jnp.pad(ptz, pad), jnp.pad(pbz, pad)], axis=0)
    g = _egather(tab3, tasks)

    yf = y.astype(jnp.float32).reshape(E, 1)
    logits, prob, wh, lt, le, la = _edge(
        g, yf, edge_attr, Wb, ba.reshape(1, -1), bb.reshape(1, -1))

    return (lt[0, 0], le[0, 0], la[0, 0], logits[:, 0], prob[:, 0], z, wh)


# speedup vs baseline: 4.1117x; 1.0874x over previous
import jax
import jax.numpy as jnp
from jax import lax
from jax.experimental import pallas as pl
from jax.experimental.pallas import tpu as pltpu
from jax.experimental.pallas import tpu_sc as plsc

N = 10000
E = 160000
D = 256
EMB = 128
DE = 16

NPAD = 10240
NC, NS = 2, 16
NW = NC * NS

SB = 200
SJ = E // NS // SB
GB = 80
GJ = 2 * E // NW // GB
KPS = 1
KPG = 5

def _sc_mesh():
    return plsc.VectorSubcoreMesh(
        core_axis_name="c", subcore_axis_name="s",
        num_cores=NC, num_subcores=NS)


def _segsum_body(tab, srcr, dstr, zeros, out, idxs, idxd, rows, shared, sems):
    c = lax.axis_index("c")
    s = lax.axis_index("s")
    pltpu.sync_copy(zeros, shared.at[pl.ds(s * 640, 640)])
    pltpu.sync_copy(srcr.at[c, s], idxs)
    pltpu.sync_copy(dstr.at[s], idxd)
    plsc.subcore_barrier()

    def body(jj, carry):
        j0 = jj * KPS
        cps = [pltpu.async_copy(
                   tab.at[idxs.at[pl.ds((j0 + k) * SB, SB)]], rows[k], sems[k])
               for k in range(KPS)]
        for k in range(KPS):
            cps[k].wait()
            pltpu.sync_copy(rows[k],
                            shared.at[idxd.at[pl.ds((j0 + k) * SB, SB)]],
                            add=True)
        return carry

    lax.fori_loop(0, SJ // KPS, body, 0)
    plsc.subcore_barrier()
    pltpu.sync_copy(shared.at[pl.ds(s * 640, 640)],
                    out.at[pl.ds(c * NPAD + s * 640, 640)])


def _segsum(tab, srcr, dstr, zeros):
    return pl.kernel(
        _segsum_body,
        out_type=jax.ShapeDtypeStruct((2 * NPAD, 128), jnp.float32),
        mesh=_sc_mesh(),
        scratch_types=[
            pltpu.VMEM((SJ * SB,), jnp.int32),
            pltpu.VMEM((SJ * SB,), jnp.int32),
            [pltpu.VMEM((SB, 128), jnp.float32)] * KPS,
            pltpu.VMEM_SHARED((NPAD, 128), jnp.float32),
            [pltpu.SemaphoreType.DMA] * KPS,
        ],
    )(tab, srcr, dstr, zeros)


PW = D // 2 + EMB


def _egather_body(tab, tasks, out, idx, rows, sems):
    w = lax.axis_index("c") * NS + lax.axis_index("s")
    pltpu.sync_copy(tasks.at[w], idx)
    base = w * (GJ * GB)

    def body(jj, carry):
        j0 = jj * KPG
        cps = [pltpu.async_copy(tab.at[idx.at[j0 + k]], rows[k], sems[k])
               for k in range(KPG)]
        for k in range(KPG):
            cps[k].wait()
            pltpu.sync_copy(rows[k],
                            out.at[pl.ds(base + (j0 + k) * GB, GB)])
        return carry

    lax.fori_loop(0, GJ // KPG, body, 0)


def _egather(tab, tasks):
    return pl.kernel(
        _egather_body,
        out_type=jax.ShapeDtypeStruct((2 * E, PW), jnp.uint32),
        mesh=_sc_mesh(),
        scratch_types=[
            pltpu.VMEM((GJ, GB), jnp.int32),
            [pltpu.VMEM((GB, PW), jnp.uint32)] * KPG,
            [pltpu.SemaphoreType.DMA] * KPG,
        ],
    )(tab, tasks)


MR = 1000
MNB = N // MR

def _mlp1_body(x_ref, al_ref, ar_ref, wx_ref, bx_ref, wy_ref, by_ref, o_ref):
    m = x_ref[...] + jnp.concatenate([al_ref[...], ar_ref[...]], axis=1)
    h = jnp.maximum(
        jnp.dot(m, wx_ref[...], preferred_element_type=jnp.float32)
        + bx_ref[...], 0.0)
    o = jnp.dot(h, wy_ref[...], preferred_element_type=jnp.float32) + by_ref[...]
    o_ref[...] = jnp.maximum(o, 0.0)


def _mlp1(x, al, ar, wx, bx, wy, by):
    return pl.pallas_call(
        _mlp1_body,
        grid=(MNB,),
        in_specs=[
            pl.BlockSpec((MR, D), lambda i: (i, 0)),
            pl.BlockSpec((MR, EMB), lambda i: (i, 0)),
            pl.BlockSpec((MR, EMB), lambda i: (i, 0)),
            pl.BlockSpec((D, D), lambda i: (0, 0)),
            pl.BlockSpec((1, D), lambda i: (0, 0)),
            pl.BlockSpec((D, D), lambda i: (0, 0)),
            pl.BlockSpec((1, D), lambda i: (0, 0)),
        ],
        out_specs=pl.BlockSpec((MR, D), lambda i: (i, 0)),
        out_shape=jax.ShapeDtypeStruct((N, D), jnp.float32),
    )(x, al, ar, wx, bx, wy, by)


def _pack(p, z):
    bits = lax.bitcast_convert_type(p, jnp.uint32) + jnp.uint32(0x8000)
    lo = bits[:, :D // 2] >> 16
    hi = bits[:, D // 2:] & jnp.uint32(0xFFFF0000)
    return jnp.concatenate(
        [lo | hi, lax.bitcast_convert_type(z, jnp.uint32)], axis=1)


def _l2_body(x_ref, al_ref, ar_ref, wx_ref, bx_ref, wy_ref, by_ref, wa_ref,
             z_ref, pt_ref, pb_ref):
    m = x_ref[...] + jnp.concatenate([al_ref[...], ar_ref[...]], axis=1)
    h = jnp.maximum(
        jnp.dot(m, wx_ref[...], preferred_element_type=jnp.float32)
        + bx_ref[...], 0.0)
    z = jnp.dot(h, wy_ref[...], preferred_element_type=jnp.float32) + by_ref[...]
    z_ref[...] = z
    wa = wa_ref[...]
    pt = jnp.dot(z, wa[:EMB, :], preferred_element_type=jnp.float32)
    pb = jnp.dot(z, wa[EMB:, :], preferred_element_type=jnp.float32)
    pt_ref[...] = _pack(pt, z)
    pb_ref[...] = _pack(pb, z)


def _l2(x, al, ar, wx, bx, wy, by, wa):
    return pl.pallas_call(
        _l2_body,
        grid=(MNB,),
        in_specs=[
            pl.BlockSpec((MR, D), lambda i: (i, 0)),
            pl.BlockSpec((MR, EMB), lambda i: (i, 0)),
            pl.BlockSpec((MR, EMB), lambda i: (i, 0)),
            pl.BlockSpec((D, D), lambda i: (0, 0)),
            pl.BlockSpec((1, D), lambda i: (0, 0)),
            pl.BlockSpec((D, EMB), lambda i: (0, 0)),
            pl.BlockSpec((1, EMB), lambda i: (0, 0)),
            pl.BlockSpec((D, D), lambda i: (0, 0)),
        ],
        out_specs=[
            pl.BlockSpec((MR, EMB), lambda i: (i, 0)),
            pl.BlockSpec((MR, PW), lambda i: (i, 0)),
            pl.BlockSpec((MR, PW), lambda i: (i, 0)),
        ],
        out_shape=[
            jax.ShapeDtypeStruct((N, EMB), jnp.float32),
            jax.ShapeDtypeStruct((N, PW), jnp.uint32),
            jax.ShapeDtypeStruct((N, PW), jnp.uint32),
        ],
    )(x, al, ar, wx, bx, wy, by, wa)


EB = 2000
ENB = E // EB


def _unpack(w):
    wp = w[:, :D // 2]
    lo = lax.bitcast_convert_type(wp << 16, jnp.float32)
    hi = lax.bitcast_convert_type(wp & jnp.uint32(0xFFFF0000), jnp.float32)
    z = lax.bitcast_convert_type(w[:, D // 2:], jnp.float32)
    return jnp.concatenate([lo, hi], axis=1), z

def _edge_body(gs_ref, gd_ref, yf_ref, ea_ref, wb_ref, ba_ref, bb_ref,
               logit_ref, prob_ref, wh_ref, lt_ref, le_ref, la_ref, acc_ref):
    i = pl.program_id(0)
    ps, zs = _unpack(gs_ref[...])
    pd, zd = _unpack(gd_ref[...])
    hidden = jnp.maximum(ps + pd + ba_ref[...], 0.0)
    wh = (jnp.dot(hidden, wb_ref[...], preferred_element_type=jnp.float32)
          + bb_ref[...])
    wh_ref[...] = wh

    dif = zs - zd
    lane = lax.broadcasted_iota(jnp.int32, (EB, EMB), 1)
    d2 = jnp.sum(jnp.where(lane < EMB - 1, dif * dif, 0.0), axis=1,
                 keepdims=True)
    mass = zd[:, EMB - 1:EMB]
    logits = mass - jnp.log(d2 + 1e-8)
    logit_ref[...] = logits
    prob_ref[...] = jax.nn.sigmoid(logits)

    yf = yf_ref[...]
    bce = (jnp.maximum(logits, 0.0) - logits * yf
           + jnp.log1p(jnp.exp(-jnp.abs(logits))))
    bsum = jnp.sum(bce)
    asum = jnp.sum(((wh - ea_ref[...]) ** 2) * yf)
    csum = jnp.sum(yf)

    @pl.when(i == 0)
    def _():
        acc_ref[0] = bsum
        acc_ref[1] = asum
        acc_ref[2] = csum

    @pl.when(i > 0)
    def _():
        acc_ref[0] = acc_ref[0] + bsum
        acc_ref[1] = acc_ref[1] + asum
        acc_ref[2] = acc_ref[2] + csum

    @pl.when(i == ENB - 1)
    def _():
        le = acc_ref[0] / E
        la = acc_ref[1] / jnp.maximum(acc_ref[2] * DE, 1.0)
        le_ref[...] = jnp.full((1, 1), le, jnp.float32)
        la_ref[...] = jnp.full((1, 1), la, jnp.float32)
        lt_ref[...] = jnp.full((1, 1), le + la, jnp.float32)


def _edge(g, yf, ea, wb, ba, bb):
    return pl.pallas_call(
        _edge_body,
        grid=(ENB,),
        in_specs=[
            pl.BlockSpec((EB, PW), lambda i: (i, 0)),
            pl.BlockSpec((EB, PW), lambda i: (i + ENB, 0)),
            pl.BlockSpec((EB, 1), lambda i: (i, 0)),
            pl.BlockSpec((EB, DE), lambda i: (i, 0)),
            pl.BlockSpec((D, DE), lambda i: (0, 0)),
            pl.BlockSpec((1, D), lambda i: (0, 0)),
            pl.BlockSpec((1, DE), lambda i: (0, 0)),
        ],
        out_specs=[
            pl.BlockSpec((EB, 1), lambda i: (i, 0)),
            pl.BlockSpec((EB, 1), lambda i: (i, 0)),
            pl.BlockSpec((EB, DE), lambda i: (i, 0)),
            pl.BlockSpec((1, 1), lambda i: (0, 0)),
            pl.BlockSpec((1, 1), lambda i: (0, 0)),
            pl.BlockSpec((1, 1), lambda i: (0, 0)),
        ],
        out_shape=[
            jax.ShapeDtypeStruct((E, 1), jnp.float32),
            jax.ShapeDtypeStruct((E, 1), jnp.float32),
            jax.ShapeDtypeStruct((E, DE), jnp.float32),
            jax.ShapeDtypeStruct((1, 1), jnp.float32),
            jax.ShapeDtypeStruct((1, 1), jnp.float32),
            jax.ShapeDtypeStruct((1, 1), jnp.float32),
        ],
        scratch_shapes=[pltpu.SMEM((3,), jnp.float32)],
        compiler_params=pltpu.CompilerParams(
            dimension_semantics=("arbitrary",)),
    )(g, g, yf, ea, wb, ba, bb)


def _split_stack(a):
    pad = ((0, NPAD - N), (0, 0))
    return jnp.concatenate(
        [jnp.pad(a[:, :EMB], pad), jnp.pad(a[:, EMB:], pad)], axis=0)


def kernel(x, edge_index, y, edge_attr, W1a, b1a, W1b, b1b, W2a, b2a, W2b,
           b2b, Wa, ba, Wb, bb):
    src = edge_index[0]
    dst = edge_index[1]

    src_c = src.reshape(NS, SJ * SB)
    srcr = jnp.stack([src_c, src_c + NPAD])
    dstr = dst.reshape(NS, SJ * SB)
    tasks = jnp.concatenate([src, dst + NPAD]).reshape(NW, GJ, GB)
    zeros = jnp.zeros((640, EMB), jnp.float32)

    o1 = _segsum(_split_stack(x), srcr, dstr, zeros)
    h1 = _mlp1(x, o1[:N], o1[NPAD:NPAD + N],
               W1a, b1a.reshape(1, -1), W1b, b1b.reshape(1, -1))

    o2 = _segsum(_split_stack(h1), srcr, dstr, zeros)
    z, ptz, pbz = _l2(h1, o2[:N], o2[NPAD:NPAD + N],
                      W2a, b2a.reshape(1, -1), W2b, b2b.reshape(1, -1), Wa)

    pad = ((0, NPAD - N), (0, 0))
    tab3 = jnp.concatenate([jnp.pad(ptz, pad), jnp.pad(pbz, pad)], axis=0)
    g = _egather(tab3, tasks)

    yf = y.astype(jnp.float32).reshape(E, 1)
    logits, prob, wh, lt, le, la = _edge(
        g, yf, edge_attr, Wb, ba.reshape(1, -1), bb.reshape(1, -1))

    return (lt[0, 0], le[0, 0], la[0, 0], logits[:, 0], prob[:, 0], z, wh)


# speedup vs baseline: 4.1362x; 1.0060x over previous
import jax
import jax.numpy as jnp
from jax import lax
from jax.experimental import pallas as pl
from jax.experimental.pallas import tpu as pltpu
from jax.experimental.pallas import tpu_sc as plsc

N = 10000
E = 160000
D = 256
EMB = 128
DE = 16

NPAD = 10240
NC, NS = 2, 16
NW = NC * NS

SB = 200
SJ = E // NS // SB
GB = 200
GJ = 2 * E // NW // GB
KPS = 1
KPG = 2

def _sc_mesh():
    return plsc.VectorSubcoreMesh(
        core_axis_name="c", subcore_axis_name="s",
        num_cores=NC, num_subcores=NS)


def _segsum_body(tab, srcr, dstr, zeros, out, idxs, idxd, rows, shared, sems):
    c = lax.axis_index("c")
    s = lax.axis_index("s")
    pltpu.sync_copy(zeros, shared.at[pl.ds(s * 640, 640)])
    pltpu.sync_copy(srcr.at[c, s], idxs)
    pltpu.sync_copy(dstr.at[s], idxd)
    plsc.subcore_barrier()

    def body(jj, carry):
        j0 = jj * KPS
        cps = [pltpu.async_copy(
                   tab.at[idxs.at[pl.ds((j0 + k) * SB, SB)]], rows[k], sems[k])
               for k in range(KPS)]
        for k in range(KPS):
            cps[k].wait()
            pltpu.sync_copy(rows[k],
                            shared.at[idxd.at[pl.ds((j0 + k) * SB, SB)]],
                            add=True)
        return carry

    lax.fori_loop(0, SJ // KPS, body, 0)
    plsc.subcore_barrier()
    pltpu.sync_copy(shared.at[pl.ds(s * 640, 640)],
                    out.at[pl.ds(c * NPAD + s * 640, 640)])


def _segsum(tab, srcr, dstr, zeros):
    return pl.kernel(
        _segsum_body,
        out_type=jax.ShapeDtypeStruct((2 * NPAD, 128), jnp.float32),
        mesh=_sc_mesh(),
        scratch_types=[
            pltpu.VMEM((SJ * SB,), jnp.int32),
            pltpu.VMEM((SJ * SB,), jnp.int32),
            [pltpu.VMEM((SB, 128), jnp.float32)] * KPS,
            pltpu.VMEM_SHARED((NPAD, 128), jnp.float32),
            [pltpu.SemaphoreType.DMA] * KPS,
        ],
    )(tab, srcr, dstr, zeros)


PW = D // 2 + EMB


def _egather_body(tab, tasks, out, idx, rows, sems):
    w = lax.axis_index("c") * NS + lax.axis_index("s")
    pltpu.sync_copy(tasks.at[w], idx)
    base = w * (GJ * GB)

    def body(jj, carry):
        j0 = jj * KPG
        cps = [pltpu.async_copy(
                   tab.at[idx.at[pl.ds((j0 + k) * GB, GB)]], rows[k], sems[k])
               for k in range(KPG)]
        for k in range(KPG):
            cps[k].wait()
            pltpu.sync_copy(rows[k],
                            out.at[pl.ds(base + (j0 + k) * GB, GB)])
        return carry

    lax.fori_loop(0, GJ // KPG, body, 0)


def _egather(tab, tasks):
    return pl.kernel(
        _egather_body,
        out_type=jax.ShapeDtypeStruct((2 * E, PW), jnp.uint32),
        mesh=_sc_mesh(),
        scratch_types=[
            pltpu.VMEM((GJ * GB,), jnp.int32),
            [pltpu.VMEM((GB, PW), jnp.uint32)] * KPG,
            [pltpu.SemaphoreType.DMA] * KPG,
        ],
    )(tab, tasks)


MR = 1000
MNB = N // MR

def _mlp1_body(x_ref, al_ref, ar_ref, wx_ref, bx_ref, wy_ref, by_ref, o_ref):
    m = x_ref[...] + jnp.concatenate([al_ref[...], ar_ref[...]], axis=1)
    h = jnp.maximum(
        jnp.dot(m, wx_ref[...], preferred_element_type=jnp.float32)
        + bx_ref[...], 0.0)
    o = jnp.dot(h, wy_ref[...], preferred_element_type=jnp.float32) + by_ref[...]
    o_ref[...] = jnp.maximum(o, 0.0)


def _mlp1(x, al, ar, wx, bx, wy, by):
    return pl.pallas_call(
        _mlp1_body,
        grid=(MNB,),
        in_specs=[
            pl.BlockSpec((MR, D), lambda i: (i, 0)),
            pl.BlockSpec((MR, EMB), lambda i: (i, 0)),
            pl.BlockSpec((MR, EMB), lambda i: (i, 0)),
            pl.BlockSpec((D, D), lambda i: (0, 0)),
            pl.BlockSpec((1, D), lambda i: (0, 0)),
            pl.BlockSpec((D, D), lambda i: (0, 0)),
            pl.BlockSpec((1, D), lambda i: (0, 0)),
        ],
        out_specs=pl.BlockSpec((MR, D), lambda i: (i, 0)),
        out_shape=jax.ShapeDtypeStruct((N, D), jnp.float32),
    )(x, al, ar, wx, bx, wy, by)


def _pack(p, z):
    bits = lax.bitcast_convert_type(p, jnp.uint32) + jnp.uint32(0x8000)
    lo = bits[:, :D // 2] >> 16
    hi = bits[:, D // 2:] & jnp.uint32(0xFFFF0000)
    return jnp.concatenate(
        [lo | hi, lax.bitcast_convert_type(z, jnp.uint32)], axis=1)


def _l2_body(x_ref, al_ref, ar_ref, wx_ref, bx_ref, wy_ref, by_ref, wa_ref,
             z_ref, pt_ref, pb_ref):
    m = x_ref[...] + jnp.concatenate([al_ref[...], ar_ref[...]], axis=1)
    h = jnp.maximum(
        jnp.dot(m, wx_ref[...], preferred_element_type=jnp.float32)
        + bx_ref[...], 0.0)
    z = jnp.dot(h, wy_ref[...], preferred_element_type=jnp.float32) + by_ref[...]
    z_ref[...] = z
    wa = wa_ref[...]
    pt = jnp.dot(z, wa[:EMB, :], preferred_element_type=jnp.float32)
    pb = jnp.dot(z, wa[EMB:, :], preferred_element_type=jnp.float32)
    pt_ref[...] = _pack(pt, z)
    pb_ref[...] = _pack(pb, z)


def _l2(x, al, ar, wx, bx, wy, by, wa):
    return pl.pallas_call(
        _l2_body,
        grid=(MNB,),
        in_specs=[
            pl.BlockSpec((MR, D), lambda i: (i, 0)),
            pl.BlockSpec((MR, EMB), lambda i: (i, 0)),
            pl.BlockSpec((MR, EMB), lambda i: (i, 0)),
            pl.BlockSpec((D, D), lambda i: (0, 0)),
            pl.BlockSpec((1, D), lambda i: (0, 0)),
            pl.BlockSpec((D, EMB), lambda i: (0, 0)),
            pl.BlockSpec((1, EMB), lambda i: (0, 0)),
            pl.BlockSpec((D, D), lambda i: (0, 0)),
        ],
        out_specs=[
            pl.BlockSpec((MR, EMB), lambda i: (i, 0)),
            pl.BlockSpec((MR, PW), lambda i: (i, 0)),
            pl.BlockSpec((MR, PW), lambda i: (i, 0)),
        ],
        out_shape=[
            jax.ShapeDtypeStruct((N, EMB), jnp.float32),
            jax.ShapeDtypeStruct((N, PW), jnp.uint32),
            jax.ShapeDtypeStruct((N, PW), jnp.uint32),
        ],
    )(x, al, ar, wx, bx, wy, by, wa)


EB = 2000
ENB = E // EB


def _unpack(w):
    wp = w[:, :D // 2]
    lo = lax.bitcast_convert_type(wp << 16, jnp.float32)
    hi = lax.bitcast_convert_type(wp & jnp.uint32(0xFFFF0000), jnp.float32)
    z = lax.bitcast_convert_type(w[:, D // 2:], jnp.float32)
    return jnp.concatenate([lo, hi], axis=1), z

def _edge_body(gs_ref, gd_ref, yf_ref, ea_ref, wb_ref, ba_ref, bb_ref,
               logit_ref, prob_ref, wh_ref, lt_ref, le_ref, la_ref, acc_ref):
    i = pl.program_id(0)
    ps, zs = _unpack(gs_ref[...])
    pd, zd = _unpack(gd_ref[...])
    hidden = jnp.maximum(ps + pd + ba_ref[...], 0.0)
    wh = (jnp.dot(hidden, wb_ref[...], preferred_element_type=jnp.float32)
          + bb_ref[...])
    wh_ref[...] = wh

    dif = zs - zd
    lane = lax.broadcasted_iota(jnp.int32, (EB, EMB), 1)
    d2 = jnp.sum(jnp.where(lane < EMB - 1, dif * dif, 0.0), axis=1,
                 keepdims=True)
    mass = zd[:, EMB - 1:EMB]
    logits = mass - jnp.log(d2 + 1e-8)
    logit_ref[...] = logits
    prob_ref[...] = jax.nn.sigmoid(logits)

    yf = yf_ref[...]
    bce = (jnp.maximum(logits, 0.0) - logits * yf
           + jnp.log1p(jnp.exp(-jnp.abs(logits))))
    bsum = jnp.sum(bce)
    asum = jnp.sum(((wh - ea_ref[...]) ** 2) * yf)
    csum = jnp.sum(yf)

    @pl.when(i == 0)
    def _():
        acc_ref[0] = bsum
        acc_ref[1] = asum
        acc_ref[2] = csum

    @pl.when(i > 0)
    def _():
        acc_ref[0] = acc_ref[0] + bsum
        acc_ref[1] = acc_ref[1] + asum
        acc_ref[2] = acc_ref[2] + csum

    @pl.when(i == ENB - 1)
    def _():
        le = acc_ref[0] / E
        la = acc_ref[1] / jnp.maximum(acc_ref[2] * DE, 1.0)
        le_ref[...] = jnp.full((1, 1), le, jnp.float32)
        la_ref[...] = jnp.full((1, 1), la, jnp.float32)
        lt_ref[...] = jnp.full((1, 1), le + la, jnp.float32)


def _edge(g, yf, ea, wb, ba, bb):
    return pl.pallas_call(
        _edge_body,
        grid=(ENB,),
        in_specs=[
            pl.BlockSpec((EB, PW), lambda i: (i, 0)),
            pl.BlockSpec((EB, PW), lambda i: (i + ENB, 0)),
            pl.BlockSpec((EB, 1), lambda i: (i, 0)),
            pl.BlockSpec((EB, DE), lambda i: (i, 0)),
            pl.BlockSpec((D, DE), lambda i: (0, 0)),
            pl.BlockSpec((1, D), lambda i: (0, 0)),
            pl.BlockSpec((1, DE), lambda i: (0, 0)),
        ],
        out_specs=[
            pl.BlockSpec((EB, 1), lambda i: (i, 0)),
            pl.BlockSpec((EB, 1), lambda i: (i, 0)),
            pl.BlockSpec((EB, DE), lambda i: (i, 0)),
            pl.BlockSpec((1, 1), lambda i: (0, 0)),
            pl.BlockSpec((1, 1), lambda i: (0, 0)),
            pl.BlockSpec((1, 1), lambda i: (0, 0)),
        ],
        out_shape=[
            jax.ShapeDtypeStruct((E, 1), jnp.float32),
            jax.ShapeDtypeStruct((E, 1), jnp.float32),
            jax.ShapeDtypeStruct((E, DE), jnp.float32),
            jax.ShapeDtypeStruct((1, 1), jnp.float32),
            jax.ShapeDtypeStruct((1, 1), jnp.float32),
            jax.ShapeDtypeStruct((1, 1), jnp.float32),
        ],
        scratch_shapes=[pltpu.SMEM((3,), jnp.float32)],
        compiler_params=pltpu.CompilerParams(
            dimension_semantics=("arbitrary",)),
    )(g, g, yf, ea, wb, ba, bb)


def _split_stack(a):
    pad = ((0, NPAD - N), (0, 0))
    return jnp.concatenate(
        [jnp.pad(a[:, :EMB], pad), jnp.pad(a[:, EMB:], pad)], axis=0)


def kernel(x, edge_index, y, edge_attr, W1a, b1a, W1b, b1b, W2a, b2a, W2b,
           b2b, Wa, ba, Wb, bb):
    src = edge_index[0]
    dst = edge_index[1]

    src_c = src.reshape(NS, SJ * SB)
    srcr = jnp.stack([src_c, src_c + NPAD])
    dstr = dst.reshape(NS, SJ * SB)
    tasks = jnp.concatenate([src, dst + NPAD]).reshape(NW, GJ * GB)
    zeros = jnp.zeros((640, EMB), jnp.float32)

    o1 = _segsum(_split_stack(x), srcr, dstr, zeros)
    h1 = _mlp1(x, o1[:N], o1[NPAD:NPAD + N],
               W1a, b1a.reshape(1, -1), W1b, b1b.reshape(1, -1))

    o2 = _segsum(_split_stack(h1), srcr, dstr, zeros)
    z, ptz, pbz = _l2(h1, o2[:N], o2[NPAD:NPAD + N],
                      W2a, b2a.reshape(1, -1), W2b, b2b.reshape(1, -1), Wa)

    pad = ((0, NPAD - N), (0, 0))
    tab3 = jnp.concatenate([jnp.pad(ptz, pad), jnp.pad(pbz, pad)], axis=0)
    g = _egather(tab3, tasks)

    yf = y.astype(jnp.float32).reshape(E, 1)
    logits, prob, wh, lt, le, la = _edge(
        g, yf, edge_attr, Wb, ba.reshape(1, -1), bb.reshape(1, -1))

    return (lt[0, 0], le[0, 0], la[0, 0], logits[:, 0], prob[:, 0], z, wh)


# speedup vs baseline: 4.2222x; 1.0208x over previous
import jax
import jax.numpy as jnp
from jax import lax
from jax.experimental import pallas as pl
from jax.experimental.pallas import tpu as pltpu
from jax.experimental.pallas import tpu_sc as plsc

N = 10000
E = 160000
D = 256
EMB = 128
DE = 16

NPAD = 10240
NC, NS = 2, 16
NW = NC * NS

SB = 200
SJ = E // NS // SB
GB = 200
GJ = 2 * E // NW // GB
KPS = 1
KPG = 2

def _sc_mesh():
    return plsc.VectorSubcoreMesh(
        core_axis_name="c", subcore_axis_name="s",
        num_cores=NC, num_subcores=NS)


def _segsum_body(tab, srcr, dstr, zeros, out, idxs, idxd, rows, shared, sems):
    c = lax.axis_index("c")
    s = lax.axis_index("s")
    pltpu.sync_copy(zeros, shared.at[pl.ds(s * 640, 640)])
    pltpu.sync_copy(srcr.at[s], idxs)
    pltpu.sync_copy(dstr.at[s], idxd)
    plsc.subcore_barrier()

    def body(jj, carry):
        j0 = jj * KPS
        cps = [pltpu.async_copy(
                   tab.at[idxs.at[pl.ds((j0 + k) * SB, SB)],
                          pl.ds(c * 128, 128)],
                   rows[k], sems[k])
               for k in range(KPS)]
        for k in range(KPS):
            cps[k].wait()
            pltpu.sync_copy(rows[k],
                            shared.at[idxd.at[pl.ds((j0 + k) * SB, SB)]],
                            add=True)
        return carry

    lax.fori_loop(0, SJ // KPS, body, 0)
    plsc.subcore_barrier()
    pltpu.sync_copy(shared.at[pl.ds(s * 640, 640)],
                    out.at[pl.ds(c * NPAD + s * 640, 640)])


def _segsum(tab, srcr, dstr, zeros):
    return pl.kernel(
        _segsum_body,
        out_type=jax.ShapeDtypeStruct((2 * NPAD, 128), jnp.float32),
        mesh=_sc_mesh(),
        scratch_types=[
            pltpu.VMEM((SJ * SB,), jnp.int32),
            pltpu.VMEM((SJ * SB,), jnp.int32),
            [pltpu.VMEM((SB, 128), jnp.float32)] * KPS,
            pltpu.VMEM_SHARED((NPAD, 128), jnp.float32),
            [pltpu.SemaphoreType.DMA] * KPS,
        ],
    )(tab, srcr, dstr, zeros)


PW = D // 2 + EMB


def _egather_body(tab, tasks, out, idx, rows, sems):
    w = lax.axis_index("c") * NS + lax.axis_index("s")
    pltpu.sync_copy(tasks.at[w], idx)
    base = w * (GJ * GB)

    def body(jj, carry):
        j0 = jj * KPG
        cps = [pltpu.async_copy(
                   tab.at[idx.at[pl.ds((j0 + k) * GB, GB)]], rows[k], sems[k])
               for k in range(KPG)]
        for k in range(KPG):
            cps[k].wait()
            pltpu.sync_copy(rows[k],
                            out.at[pl.ds(base + (j0 + k) * GB, GB)])
        return carry

    lax.fori_loop(0, GJ // KPG, body, 0)


def _egather(tab, tasks):
    return pl.kernel(
        _egather_body,
        out_type=jax.ShapeDtypeStruct((2 * E, PW), jnp.uint32),
        mesh=_sc_mesh(),
        scratch_types=[
            pltpu.VMEM((GJ * GB,), jnp.int32),
            [pltpu.VMEM((GB, PW), jnp.uint32)] * KPG,
            [pltpu.SemaphoreType.DMA] * KPG,
        ],
    )(tab, tasks)


MR = 1000
MNB = N // MR

def _mlp1_body(x_ref, al_ref, ar_ref, wx_ref, bx_ref, wy_ref, by_ref, o_ref):
    m = x_ref[...] + jnp.concatenate([al_ref[...], ar_ref[...]], axis=1)
    h = jnp.maximum(
        jnp.dot(m, wx_ref[...], preferred_element_type=jnp.float32)
        + bx_ref[...], 0.0)
    o = jnp.dot(h, wy_ref[...], preferred_element_type=jnp.float32) + by_ref[...]
    o_ref[...] = jnp.maximum(o, 0.0)


def _mlp1(x, al, ar, wx, bx, wy, by):
    return pl.pallas_call(
        _mlp1_body,
        grid=(MNB,),
        in_specs=[
            pl.BlockSpec((MR, D), lambda i: (i, 0)),
            pl.BlockSpec((MR, EMB), lambda i: (i, 0)),
            pl.BlockSpec((MR, EMB), lambda i: (i, 0)),
            pl.BlockSpec((D, D), lambda i: (0, 0)),
            pl.BlockSpec((1, D), lambda i: (0, 0)),
            pl.BlockSpec((D, D), lambda i: (0, 0)),
            pl.BlockSpec((1, D), lambda i: (0, 0)),
        ],
        out_specs=pl.BlockSpec((MR, D), lambda i: (i, 0)),
        out_shape=jax.ShapeDtypeStruct((N, D), jnp.float32),
    )(x, al, ar, wx, bx, wy, by)


def _pack(p, z):
    bits = lax.bitcast_convert_type(p, jnp.uint32) + jnp.uint32(0x8000)
    lo = bits[:, :D // 2] >> 16
    hi = bits[:, D // 2:] & jnp.uint32(0xFFFF0000)
    return jnp.concatenate(
        [lo | hi, lax.bitcast_convert_type(z, jnp.uint32)], axis=1)


def _l2_body(x_ref, al_ref, ar_ref, wx_ref, bx_ref, wy_ref, by_ref, wa_ref,
             z_ref, pt_ref, pb_ref):
    m = x_ref[...] + jnp.concatenate([al_ref[...], ar_ref[...]], axis=1)
    h = jnp.maximum(
        jnp.dot(m, wx_ref[...], preferred_element_type=jnp.float32)
        + bx_ref[...], 0.0)
    z = jnp.dot(h, wy_ref[...], preferred_element_type=jnp.float32) + by_ref[...]
    z_ref[...] = z
    wa = wa_ref[...]
    pt = jnp.dot(z, wa[:EMB, :], preferred_element_type=jnp.float32)
    pb = jnp.dot(z, wa[EMB:, :], preferred_element_type=jnp.float32)
    pt_ref[...] = _pack(pt, z)
    pb_ref[...] = _pack(pb, z)


def _l2(x, al, ar, wx, bx, wy, by, wa):
    return pl.pallas_call(
        _l2_body,
        grid=(MNB,),
        in_specs=[
            pl.BlockSpec((MR, D), lambda i: (i, 0)),
            pl.BlockSpec((MR, EMB), lambda i: (i, 0)),
            pl.BlockSpec((MR, EMB), lambda i: (i, 0)),
            pl.BlockSpec((D, D), lambda i: (0, 0)),
            pl.BlockSpec((1, D), lambda i: (0, 0)),
            pl.BlockSpec((D, EMB), lambda i: (0, 0)),
            pl.BlockSpec((1, EMB), lambda i: (0, 0)),
            pl.BlockSpec((D, D), lambda i: (0, 0)),
        ],
        out_specs=[
            pl.BlockSpec((MR, EMB), lambda i: (i, 0)),
            pl.BlockSpec((MR, PW), lambda i: (i, 0)),
            pl.BlockSpec((MR, PW), lambda i: (i, 0)),
        ],
        out_shape=[
            jax.ShapeDtypeStruct((N, EMB), jnp.float32),
            jax.ShapeDtypeStruct((N, PW), jnp.uint32),
            jax.ShapeDtypeStruct((N, PW), jnp.uint32),
        ],
    )(x, al, ar, wx, bx, wy, by, wa)


EB = 2000
ENB = E // EB


def _unpack(w):
    wp = w[:, :D // 2]
    lo = lax.bitcast_convert_type(wp << 16, jnp.float32)
    hi = lax.bitcast_convert_type(wp & jnp.uint32(0xFFFF0000), jnp.float32)
    z = lax.bitcast_convert_type(w[:, D // 2:], jnp.float32)
    return jnp.concatenate([lo, hi], axis=1), z

def _edge_body(gs_ref, gd_ref, yf_ref, ea_ref, wb_ref, ba_ref, bb_ref,
               logit_ref, prob_ref, wh_ref, lt_ref, le_ref, la_ref, acc_ref):
    i = pl.program_id(0)
    ps, zs = _unpack(gs_ref[...])
    pd, zd = _unpack(gd_ref[...])
    hidden = jnp.maximum(ps + pd + ba_ref[...], 0.0)
    wh = (jnp.dot(hidden, wb_ref[...], preferred_element_type=jnp.float32)
          + bb_ref[...])
    wh_ref[...] = wh

    dif = zs - zd
    lane = lax.broadcasted_iota(jnp.int32, (EB, EMB), 1)
    d2 = jnp.sum(jnp.where(lane < EMB - 1, dif * dif, 0.0), axis=1,
                 keepdims=True)
    mass = zd[:, EMB - 1:EMB]
    logits = mass - jnp.log(d2 + 1e-8)
    logit_ref[...] = logits
    prob_ref[...] = jax.nn.sigmoid(logits)

    yf = yf_ref[...]
    bce = (jnp.maximum(logits, 0.0) - logits * yf
           + jnp.log1p(jnp.exp(-jnp.abs(logits))))
    bsum = jnp.sum(bce)
    asum = jnp.sum(((wh - ea_ref[...]) ** 2) * yf)
    csum = jnp.sum(yf)

    @pl.when(i == 0)
    def _():
        acc_ref[0] = bsum
        acc_ref[1] = asum
        acc_ref[2] = csum

    @pl.when(i > 0)
    def _():
        acc_ref[0] = acc_ref[0] + bsum
        acc_ref[1] = acc_ref[1] + asum
        acc_ref[2] = acc_ref[2] + csum

    @pl.when(i == ENB - 1)
    def _():
        le = acc_ref[0] / E
        la = acc_ref[1] / jnp.maximum(acc_ref[2] * DE, 1.0)
        le_ref[...] = jnp.full((1, 1), le, jnp.float32)
        la_ref[...] = jnp.full((1, 1), la, jnp.float32)
        lt_ref[...] = jnp.full((1, 1), le + la, jnp.float32)


def _edge(g, yf, ea, wb, ba, bb):
    return pl.pallas_call(
        _edge_body,
        grid=(ENB,),
        in_specs=[
            pl.BlockSpec((EB, PW), lambda i: (i, 0)),
            pl.BlockSpec((EB, PW), lambda i: (i + ENB, 0)),
            pl.BlockSpec((EB, 1), lambda i: (i, 0)),
            pl.BlockSpec((EB, DE), lambda i: (i, 0)),
            pl.BlockSpec((D, DE), lambda i: (0, 0)),
            pl.BlockSpec((1, D), lambda i: (0, 0)),
            pl.BlockSpec((1, DE), lambda i: (0, 0)),
        ],
        out_specs=[
            pl.BlockSpec((EB, 1), lambda i: (i, 0)),
            pl.BlockSpec((EB, 1), lambda i: (i, 0)),
            pl.BlockSpec((EB, DE), lambda i: (i, 0)),
            pl.BlockSpec((1, 1), lambda i: (0, 0)),
            pl.BlockSpec((1, 1), lambda i: (0, 0)),
            pl.BlockSpec((1, 1), lambda i: (0, 0)),
        ],
        out_shape=[
            jax.ShapeDtypeStruct((E, 1), jnp.float32),
            jax.ShapeDtypeStruct((E, 1), jnp.float32),
            jax.ShapeDtypeStruct((E, DE), jnp.float32),
            jax.ShapeDtypeStruct((1, 1), jnp.float32),
            jax.ShapeDtypeStruct((1, 1), jnp.float32),
            jax.ShapeDtypeStruct((1, 1), jnp.float32),
        ],
        scratch_shapes=[pltpu.SMEM((3,), jnp.float32)],
        compiler_params=pltpu.CompilerParams(
            dimension_semantics=("arbitrary",)),
    )(g, g, yf, ea, wb, ba, bb)


def kernel(x, edge_index, y, edge_attr, W1a, b1a, W1b, b1b, W2a, b2a, W2b,
           b2b, Wa, ba, Wb, bb):
    src = edge_index[0]
    dst = edge_index[1]

    srcr = src.reshape(NS, SJ * SB)
    dstr = dst.reshape(NS, SJ * SB)
    tasks = jnp.concatenate([src, dst + NPAD]).reshape(NW, GJ * GB)
    zeros = jnp.zeros((640, EMB), jnp.float32)

    o1 = _segsum(x, srcr, dstr, zeros)
    h1 = _mlp1(x, o1[:N], o1[NPAD:NPAD + N],
               W1a, b1a.reshape(1, -1), W1b, b1b.reshape(1, -1))

    o2 = _segsum(h1, srcr, dstr, zeros)
    z, ptz, pbz = _l2(h1, o2[:N], o2[NPAD:NPAD + N],
                      W2a, b2a.reshape(1, -1), W2b, b2b.reshape(1, -1), Wa)

    pad = ((0, NPAD - N), (0, 0))
    tab3 = jnp.concatenate([jnp.pad(ptz, pad), jnp.pad(pbz, pad)], axis=0)
    g = _egather(tab3, tasks)

    yf = y.astype(jnp.float32).reshape(E, 1)
    logits, prob, wh, lt, le, la = _edge(
        g, yf, edge_attr, Wb, ba.reshape(1, -1), bb.reshape(1, -1))

    return (lt[0, 0], le[0, 0], la[0, 0], logits[:, 0], prob[:, 0], z, wh)
